# Initial kernel scaffold; baseline (speedup 1.0000x reference)
import jax
import jax.numpy as jnp
from jax.experimental import pallas as pl
from jax.experimental.pallas import tpu as pltpu

_EMB = 1024
_E = 8
_TOP = 2
_W_IMP = 0.01


def _gate_body(q_ref, wv_ref, bv_ref, wo_ref, bo_ref, gw_ref, gb_ref,
               prob_ref, wcomb_ref, imp_ref, loss_ref):
    i = pl.program_id(0)
    v = jnp.dot(q_ref[...], wv_ref[...], preferred_element_type=jnp.float32)
    v = v + bv_ref[...]
    att = jnp.dot(v, wo_ref[...], preferred_element_type=jnp.float32)
    att = att + bo_ref[...]
    logits = jnp.dot(att, gw_ref[...], preferred_element_type=jnp.float32)
    logits = logits + gb_ref[...]
    m = jnp.max(logits, axis=-1, keepdims=True)
    ex = jnp.exp(logits - m)
    probs = ex / jnp.sum(ex, axis=-1, keepdims=True)
    prob_ref[...] = probs

    lane = jax.lax.broadcasted_iota(jnp.int32, probs.shape, 1)
    p1 = jnp.max(probs, axis=-1, keepdims=True)
    i1 = jnp.argmax(probs, axis=-1)
    oh1 = lane == i1[:, None]
    masked = jnp.where(oh1, -jnp.inf, probs)
    p2 = jnp.max(masked, axis=-1, keepdims=True)
    i2 = jnp.argmax(masked, axis=-1)
    oh2 = lane == i2[:, None]
    ed = jnp.exp(p2 - p1)
    w1 = 1.0 / (1.0 + ed)
    w2 = ed / (1.0 + ed)
    wcomb_ref[...] = jnp.where(oh1, w1, 0.0) + jnp.where(oh2, w2, 0.0)

    @pl.when(i == 0)
    def _init():
        imp_ref[...] = jnp.zeros_like(imp_ref)

    imp_ref[...] += jnp.sum(probs, axis=0, keepdims=True)

    @pl.when(i == pl.num_programs(0) - 1)
    def _fin():
        imp = imp_ref[0, :]
        mean = jnp.mean(imp)
        var = jnp.sum((imp - mean) ** 2) / (_E - 1)
        loss_ref[0, 0] = _W_IMP * var / (mean * mean)


def _expert_body(xf_ref, ew1_ref, eb1_ref, ew2_ref, eb2_ref, wc_ref, y_ref):
    e = pl.program_id(0)
    x = xf_ref[...]
    h = jnp.dot(x, ew1_ref[0], preferred_element_type=jnp.float32)
    h = jnp.maximum(h + eb1_ref[...], 0.0)
    out = jnp.dot(h, ew2_ref[0], preferred_element_type=jnp.float32)
    out = out + eb2_ref[...]
    onehot = (jax.lax.broadcasted_iota(jnp.int32, (_E, 1), 0) == e).astype(jnp.float32)
    w = jnp.dot(wc_ref[...], onehot, preferred_element_type=jnp.float32)

    @pl.when(e == 0)
    def _init():
        y_ref[...] = jnp.zeros_like(y_ref)

    y_ref[...] += out * w


def kernel(x, q, Wq, bq, Wk, bk, Wv, bv, Wo, bo, gate_W, gate_b, ew1, eb1, ew2, eb2):
    x_shape = x.shape
    xf = x.reshape(-1, x_shape[-1])
    n = xf.shape[0]
    tb = 512
    grid_t = n // tb

    probs, wcomb, _imp, loss = pl.pallas_call(
        _gate_body,
        grid=(grid_t,),
        in_specs=[
            pl.BlockSpec((tb, _EMB), lambda i: (i, 0)),
            pl.BlockSpec((_EMB, _EMB), lambda i: (0, 0)),
            pl.BlockSpec((1, _EMB), lambda i: (0, 0)),
            pl.BlockSpec((_EMB, _EMB), lambda i: (0, 0)),
            pl.BlockSpec((1, _EMB), lambda i: (0, 0)),
            pl.BlockSpec((_EMB, _E), lambda i: (0, 0)),
            pl.BlockSpec((1, _E), lambda i: (0, 0)),
        ],
        out_specs=[
            pl.BlockSpec((tb, _E), lambda i: (i, 0)),
            pl.BlockSpec((tb, _E), lambda i: (i, 0)),
            pl.BlockSpec((1, _E), lambda i: (0, 0)),
            pl.BlockSpec((1, 1), lambda i: (0, 0)),
        ],
        out_shape=[
            jax.ShapeDtypeStruct((n, _E), jnp.float32),
            jax.ShapeDtypeStruct((n, _E), jnp.float32),
            jax.ShapeDtypeStruct((1, _E), jnp.float32),
            jax.ShapeDtypeStruct((1, 1), jnp.float32),
        ],
    )(q, Wv, bv.reshape(1, _EMB), Wo, bo.reshape(1, _EMB),
      gate_W, gate_b.reshape(1, _E))

    y = pl.pallas_call(
        _expert_body,
        grid=(_E,),
        in_specs=[
            pl.BlockSpec((n, _EMB), lambda e: (0, 0)),
            pl.BlockSpec((1, _EMB, _EMB), lambda e: (e, 0, 0)),
            pl.BlockSpec((1, _EMB), lambda e: (e, 0)),
            pl.BlockSpec((1, _EMB, _EMB), lambda e: (e, 0, 0)),
            pl.BlockSpec((1, _EMB), lambda e: (e, 0)),
            pl.BlockSpec((n, _E), lambda e: (0, 0)),
        ],
        out_specs=pl.BlockSpec((n, _EMB), lambda e: (0, 0)),
        out_shape=jax.ShapeDtypeStruct((n, _EMB), jnp.float32),
    )(xf, ew1, eb1, ew2, eb2, wcomb)

    return (y.reshape(x_shape), probs, loss.reshape(()))


# speedup vs baseline: 3.2456x; 3.2456x over previous
import jax
import jax.numpy as jnp
from jax.experimental import pallas as pl
from jax.experimental.pallas import tpu as pltpu

_EMB = 1024
_E = 8
_TOP = 2
_W_IMP = 0.01


def _gate_body(q_ref, wv_ref, bv_ref, wo_ref, bo_ref, gw_ref, gb_ref,
               prob_ref, wcomb_ref, imp_ref, loss_ref):
    i = pl.program_id(0)
    v = jnp.dot(q_ref[...], wv_ref[...], preferred_element_type=jnp.float32)
    v = v + bv_ref[...]
    att = jnp.dot(v, wo_ref[...], preferred_element_type=jnp.float32)
    att = att + bo_ref[...]
    logits = jnp.dot(att, gw_ref[...], preferred_element_type=jnp.float32)
    logits = logits + gb_ref[...]
    m = jnp.max(logits, axis=-1, keepdims=True)
    ex = jnp.exp(logits - m)
    probs = ex / jnp.sum(ex, axis=-1, keepdims=True)
    prob_ref[...] = probs

    lane = jax.lax.broadcasted_iota(jnp.int32, probs.shape, 1)
    p1 = jnp.max(probs, axis=-1, keepdims=True)
    i1 = jnp.argmax(probs, axis=-1)
    oh1 = lane == i1[:, None]
    masked = jnp.where(oh1, -jnp.inf, probs)
    p2 = jnp.max(masked, axis=-1, keepdims=True)
    i2 = jnp.argmax(masked, axis=-1)
    oh2 = lane == i2[:, None]
    ed = jnp.exp(p2 - p1)
    w1 = 1.0 / (1.0 + ed)
    w2 = ed / (1.0 + ed)
    wcomb_ref[...] = jnp.where(oh1, w1, 0.0) + jnp.where(oh2, w2, 0.0)

    @pl.when(i == 0)
    def _init():
        imp_ref[...] = jnp.zeros_like(imp_ref)

    imp_ref[...] += jnp.sum(probs, axis=0, keepdims=True)

    @pl.when(i == pl.num_programs(0) - 1)
    def _fin():
        imp = imp_ref[0, :]
        mean = jnp.mean(imp)
        var = jnp.sum((imp - mean) ** 2) / (_E - 1)
        loss_ref[...] = jnp.broadcast_to(_W_IMP * var / (mean * mean), (1, 1))


def _expert_body(xf_ref, ew1_ref, eb1_ref, ew2_ref, eb2_ref, wc_ref, y_ref):
    e = pl.program_id(0)
    x = xf_ref[...]
    h = jnp.dot(x, ew1_ref[0], preferred_element_type=jnp.float32)
    h = jnp.maximum(h + eb1_ref[0], 0.0)
    out = jnp.dot(h, ew2_ref[0], preferred_element_type=jnp.float32)
    out = out + eb2_ref[0]
    onehot = (jax.lax.broadcasted_iota(jnp.int32, (_E, 1), 0) == e).astype(jnp.float32)
    w = jnp.dot(wc_ref[...], onehot, preferred_element_type=jnp.float32)

    @pl.when(e == 0)
    def _init():
        y_ref[...] = jnp.zeros_like(y_ref)

    y_ref[...] += out * w


def kernel(x, q, Wq, bq, Wk, bk, Wv, bv, Wo, bo, gate_W, gate_b, ew1, eb1, ew2, eb2):
    x_shape = x.shape
    xf = x.reshape(-1, x_shape[-1])
    n = xf.shape[0]
    tb = 512
    grid_t = n // tb

    probs, wcomb, _imp, loss = pl.pallas_call(
        _gate_body,
        grid=(grid_t,),
        in_specs=[
            pl.BlockSpec((tb, _EMB), lambda i: (i, 0)),
            pl.BlockSpec((_EMB, _EMB), lambda i: (0, 0)),
            pl.BlockSpec((1, _EMB), lambda i: (0, 0)),
            pl.BlockSpec((_EMB, _EMB), lambda i: (0, 0)),
            pl.BlockSpec((1, _EMB), lambda i: (0, 0)),
            pl.BlockSpec((_EMB, _E), lambda i: (0, 0)),
            pl.BlockSpec((1, _E), lambda i: (0, 0)),
        ],
        out_specs=[
            pl.BlockSpec((tb, _E), lambda i: (i, 0)),
            pl.BlockSpec((tb, _E), lambda i: (i, 0)),
            pl.BlockSpec((1, _E), lambda i: (0, 0)),
            pl.BlockSpec((1, 1), lambda i: (0, 0)),
        ],
        out_shape=[
            jax.ShapeDtypeStruct((n, _E), jnp.float32),
            jax.ShapeDtypeStruct((n, _E), jnp.float32),
            jax.ShapeDtypeStruct((1, _E), jnp.float32),
            jax.ShapeDtypeStruct((1, 1), jnp.float32),
        ],
    )(q, Wv, bv.reshape(1, _EMB), Wo, bo.reshape(1, _EMB),
      gate_W, gate_b.reshape(1, _E))

    y = pl.pallas_call(
        _expert_body,
        grid=(_E,),
        in_specs=[
            pl.BlockSpec((n, _EMB), lambda e: (0, 0)),
            pl.BlockSpec((1, _EMB, _EMB), lambda e: (e, 0, 0)),
            pl.BlockSpec((1, 1, _EMB), lambda e: (e, 0, 0)),
            pl.BlockSpec((1, _EMB, _EMB), lambda e: (e, 0, 0)),
            pl.BlockSpec((1, 1, _EMB), lambda e: (e, 0, 0)),
            pl.BlockSpec((n, _E), lambda e: (0, 0)),
        ],
        out_specs=pl.BlockSpec((n, _EMB), lambda e: (0, 0)),
        out_shape=jax.ShapeDtypeStruct((n, _EMB), jnp.float32),
    )(xf, ew1, eb1.reshape(_E, 1, _EMB), ew2, eb2.reshape(_E, 1, _EMB), wcomb)

    return (y.reshape(x_shape), probs, loss.reshape(()))
